# baseline (speedup 1.0000x reference)
import jax
import jax.numpy as jnp
from jax import lax
from jax.experimental import pallas as pl
from jax.experimental.pallas import tpu as pltpu
from jax.experimental.pallas import tpu_sc as plsc

N_NODES = 10000
DEGREE = 32
EMBED_DIM = 128
HALF_DIM = EMBED_DIM // 2
NUM_EDGES = N_NODES * DEGREE

NW = 32
EU_PER_W = NUM_EDGES // NW
EU_CHUNK = 400
EU_STEPS = EU_PER_W // EU_CHUNK
UR_PAD = 10240
UR_PER_W = UR_PAD // NW


def _gather_body(tab_hbm, nidx_hbm, uidx_hbm, eu_out, ur_out,
                 idx_v, rows_v, idx2_v, rows2_v, sem):
    nc = lax.axis_size("c")
    wid = lax.axis_index("s") * nc + lax.axis_index("c")
    base = pl.multiple_of(wid * EU_PER_W, 8)

    def chunk(c, carry):
        off = pl.multiple_of(base + c * EU_CHUNK, 8)
        pltpu.sync_copy(nidx_hbm.at[pl.ds(off, EU_CHUNK)], idx_v)
        pltpu.async_copy(tab_hbm.at[idx_v], rows_v, sem).wait()
        pltpu.sync_copy(rows_v, eu_out.at[pl.ds(off, EU_CHUNK)])
        return carry

    lax.fori_loop(0, EU_STEPS, chunk, 0, unroll=False)

    ubase = pl.multiple_of(wid * UR_PER_W, 8)
    pltpu.sync_copy(uidx_hbm.at[pl.ds(ubase, UR_PER_W)], idx2_v)
    pltpu.async_copy(tab_hbm.at[idx2_v], rows2_v, sem).wait()
    pltpu.sync_copy(rows2_v, ur_out.at[pl.ds(ubase, UR_PER_W)])


def _sc_gather(tab_i32, neigh_idx, node_idx):
    mesh = plsc.VectorSubcoreMesh(core_axis_name="c", subcore_axis_name="s")
    f = pl.kernel(
        _gather_body,
        out_type=(
            jax.ShapeDtypeStruct((NUM_EDGES, HALF_DIM), jnp.int32),
            jax.ShapeDtypeStruct((UR_PAD, HALF_DIM), jnp.int32),
        ),
        mesh=mesh,
        scratch_types=(
            pltpu.VMEM((EU_CHUNK,), jnp.int32),
            pltpu.VMEM((EU_CHUNK, HALF_DIM), jnp.int32),
            pltpu.VMEM((UR_PER_W,), jnp.int32),
            pltpu.VMEM((UR_PER_W, HALF_DIM), jnp.int32),
            pltpu.SemaphoreType.DMA,
        ),
        compiler_params=pltpu.CompilerParams(use_tc_tiling_on_sc=False),
        name="sc_neighbor_gather",
    )
    return f(tab_i32, neigh_idx, node_idx)


BN = 200
BE = BN * DEGREE
GRID = N_NODES // BN


def _mlp_body(eu_ref, ur_ref, w1a_ref, w1b_ref, w2_ref, w3r_ref,
              b1_ref, b2_ref, out_ref):
    eu = eu_ref[...]
    t = jnp.dot(ur_ref[...], w1b_ref[...],
                preferred_element_type=jnp.float32) + b1_ref[...]
    t_exp = jnp.broadcast_to(t[:, None, :], (BN, DEGREE, EMBED_DIM))
    t_exp = t_exp.reshape(BE, EMBED_DIM)
    h1 = jnp.maximum(
        jnp.dot(eu, w1a_ref[...], preferred_element_type=jnp.float32) + t_exp,
        0.0)
    h2 = jnp.maximum(
        jnp.dot(h1.astype(jnp.bfloat16), w2_ref[...],
                preferred_element_type=jnp.float32) + b2_ref[...], 0.0)
    logit = jnp.sum(h2 * w3r_ref[...], axis=1, keepdims=True)
    p = jnp.exp(logit - jnp.max(logit))
    euf = eu.astype(jnp.float32)
    num = (euf * p).reshape(BN, DEGREE, EMBED_DIM).sum(axis=1)
    den = jnp.broadcast_to(p, (BE, EMBED_DIM))
    den = den.reshape(BN, DEGREE, EMBED_DIM).sum(axis=1)
    out_ref[...] = num / den


def _tc_mlp(eu_flat, urep, W1a, W1b, W2, w3row, b1, b2, interpret=False):
    return pl.pallas_call(
        _mlp_body,
        grid=(GRID,),
        in_specs=[
            pl.BlockSpec((BE, EMBED_DIM), lambda i: (i, 0)),
            pl.BlockSpec((BN, EMBED_DIM), lambda i: (i, 0)),
            pl.BlockSpec((EMBED_DIM, EMBED_DIM), lambda i: (0, 0)),
            pl.BlockSpec((EMBED_DIM, EMBED_DIM), lambda i: (0, 0)),
            pl.BlockSpec((EMBED_DIM, EMBED_DIM), lambda i: (0, 0)),
            pl.BlockSpec((1, EMBED_DIM), lambda i: (0, 0)),
            pl.BlockSpec((1, EMBED_DIM), lambda i: (0, 0)),
            pl.BlockSpec((1, EMBED_DIM), lambda i: (0, 0)),
        ],
        out_specs=pl.BlockSpec((BN, EMBED_DIM), lambda i: (i, 0)),
        out_shape=jax.ShapeDtypeStruct((N_NODES, EMBED_DIM), jnp.float32),
        interpret=interpret,
        name="tc_attention_mlp",
    )(eu_flat, urep, W1a, W1b, W2, w3row, b1, b2)


def kernel(nodes, to_neighs, u2e, W1, b1, W2, b2, W3, b3):
    neigh_idx = to_neighs.reshape(-1).astype(jnp.int32)
    node_idx = jnp.pad(nodes.astype(jnp.int32), (0, UR_PAD - N_NODES))
    tab_bf = u2e.astype(jnp.bfloat16)
    tab_i32 = lax.bitcast_convert_type(
        tab_bf.reshape(u2e.shape[0], HALF_DIM, 2), jnp.int32)
    eu_i32, ur_i32 = _sc_gather(tab_i32, neigh_idx, node_idx)
    eu_flat = lax.bitcast_convert_type(
        eu_i32, jnp.bfloat16).reshape(NUM_EDGES, EMBED_DIM)
    urep = lax.bitcast_convert_type(
        ur_i32, jnp.bfloat16).reshape(UR_PAD, EMBED_DIM)
    W1a = W1[:EMBED_DIM].astype(jnp.bfloat16)
    W1b = W1[EMBED_DIM:].astype(jnp.bfloat16)
    w3row = W3.reshape(1, EMBED_DIM)
    return _tc_mlp(eu_flat, urep, W1a, W1b, W2.astype(jnp.bfloat16), w3row,
                   b1.reshape(1, EMBED_DIM), b2.reshape(1, EMBED_DIM))


# speedup vs baseline: 5.4689x; 5.4689x over previous
import jax
import jax.numpy as jnp
from jax import lax
from jax.experimental import pallas as pl
from jax.experimental.pallas import tpu as pltpu
from jax.experimental.pallas import tpu_sc as plsc

N_NODES = 10000
DEGREE = 32
EMBED_DIM = 128

NSLICE = 5
SL_NODES = N_NODES // NSLICE
SL_EDGES = SL_NODES * DEGREE

NW = 32
EU_PER_W = SL_EDGES // NW
EU_CHUNK = 400
EU_STEPS = EU_PER_W // EU_CHUNK
UR_PAD = 2048
UR_PER_W = UR_PAD // NW


def _gather_body(tab_hbm, nidx_hbm, uidx_hbm, eu_out, ur_out,
                 idx_v, rows0, rows1, gsem0, gsem1, ssem0, ssem1):
    nc = lax.axis_size("c")
    wid = lax.axis_index("s") * nc + lax.axis_index("c")
    base = pl.multiple_of(wid * EU_PER_W, 8)

    pltpu.sync_copy(nidx_hbm.at[pl.ds(base, EU_PER_W)], idx_v)

    rows = (rows0, rows1)
    gsem = (gsem0, gsem1)
    ssem = (ssem0, ssem1)
    gd = {}
    sd = {}
    for c in range(EU_STEPS):
        b = c % 2
        if c >= 2:
            sd[c - 2].wait()
        gd[c] = pltpu.async_copy(
            tab_hbm.at[idx_v.at[pl.ds(c * EU_CHUNK, EU_CHUNK)]],
            rows[b], gsem[b])
        if c >= 1:
            pb = (c - 1) % 2
            gd[c - 1].wait()
            off = pl.multiple_of(base + (c - 1) * EU_CHUNK, 8)
            sd[c - 1] = pltpu.async_copy(
                rows[pb], eu_out.at[pl.ds(off, EU_CHUNK)], ssem[pb])
    c = EU_STEPS - 1
    gd[c].wait()
    off = pl.multiple_of(base + c * EU_CHUNK, 8)
    sd[c] = pltpu.async_copy(rows[c % 2], eu_out.at[pl.ds(off, EU_CHUNK)],
                             ssem[c % 2])
    sd[c - 1].wait()
    sd[c].wait()

    ubase = pl.multiple_of(wid * UR_PER_W, 8)
    pltpu.sync_copy(uidx_hbm.at[pl.ds(ubase, UR_PER_W)],
                    idx_v.at[pl.ds(0, UR_PER_W)])
    pltpu.async_copy(tab_hbm.at[idx_v.at[pl.ds(0, UR_PER_W)]],
                     rows0.at[pl.ds(0, UR_PER_W)], gsem0).wait()
    pltpu.sync_copy(rows0.at[pl.ds(0, UR_PER_W)],
                    ur_out.at[pl.ds(ubase, UR_PER_W)])


def _sc_gather(u2e, neigh_idx_s, node_idx_s):
    mesh = plsc.VectorSubcoreMesh(core_axis_name="c", subcore_axis_name="s")
    f = pl.kernel(
        _gather_body,
        out_type=(
            jax.ShapeDtypeStruct((SL_EDGES, EMBED_DIM), jnp.float32),
            jax.ShapeDtypeStruct((UR_PAD, EMBED_DIM), jnp.float32),
        ),
        mesh=mesh,
        scratch_types=(
            pltpu.VMEM((EU_PER_W,), jnp.int32),
            pltpu.VMEM((EU_CHUNK, EMBED_DIM), jnp.float32),
            pltpu.VMEM((EU_CHUNK, EMBED_DIM), jnp.float32),
            pltpu.SemaphoreType.DMA,
            pltpu.SemaphoreType.DMA,
            pltpu.SemaphoreType.DMA,
            pltpu.SemaphoreType.DMA,
        ),
        name="sc_neighbor_gather",
    )
    return f(u2e, neigh_idx_s, node_idx_s)


BN = 200
BE = BN * DEGREE
SL_GRID = SL_NODES // BN


def _mlp_body(eu_ref, ur_ref, w1a_ref, w1b_ref, w2_ref, w3r_ref,
              b1_ref, b2_ref, out_ref):
    eu = eu_ref[...]
    t = jnp.dot(ur_ref[...], w1b_ref[...],
                preferred_element_type=jnp.float32) + b1_ref[...]
    t_exp = jnp.broadcast_to(t[:, None, :], (BN, DEGREE, EMBED_DIM))
    t_exp = t_exp.reshape(BE, EMBED_DIM)
    h1 = jnp.maximum(
        jnp.dot(eu, w1a_ref[...], preferred_element_type=jnp.float32) + t_exp,
        0.0)
    h2 = jnp.maximum(
        jnp.dot(h1, w2_ref[...], preferred_element_type=jnp.float32)
        + b2_ref[...], 0.0)
    logit = jnp.sum(h2 * w3r_ref[...], axis=1, keepdims=True)
    p = jnp.exp(logit - jnp.max(logit))
    num = (eu * p).reshape(BN, DEGREE, EMBED_DIM).sum(axis=1)
    den = jnp.broadcast_to(p, (BE, EMBED_DIM))
    den = den.reshape(BN, DEGREE, EMBED_DIM).sum(axis=1)
    out_ref[...] = num / den


def _tc_mlp(eu_flat, urep, W1a, W1b, W2, w3row, b1, b2, interpret=False):
    return pl.pallas_call(
        _mlp_body,
        grid=(SL_GRID,),
        in_specs=[
            pl.BlockSpec((BE, EMBED_DIM), lambda i: (i, 0)),
            pl.BlockSpec((BN, EMBED_DIM), lambda i: (i, 0)),
            pl.BlockSpec((EMBED_DIM, EMBED_DIM), lambda i: (0, 0)),
            pl.BlockSpec((EMBED_DIM, EMBED_DIM), lambda i: (0, 0)),
            pl.BlockSpec((EMBED_DIM, EMBED_DIM), lambda i: (0, 0)),
            pl.BlockSpec((1, EMBED_DIM), lambda i: (0, 0)),
            pl.BlockSpec((1, EMBED_DIM), lambda i: (0, 0)),
            pl.BlockSpec((1, EMBED_DIM), lambda i: (0, 0)),
        ],
        out_specs=pl.BlockSpec((BN, EMBED_DIM), lambda i: (i, 0)),
        out_shape=jax.ShapeDtypeStruct((SL_NODES, EMBED_DIM), jnp.float32),
        interpret=interpret,
        name="tc_attention_mlp",
    )(eu_flat, urep, W1a, W1b, W2, w3row, b1, b2)


def kernel(nodes, to_neighs, u2e, W1, b1, W2, b2, W3, b3):
    neigh_idx = to_neighs.reshape(-1).astype(jnp.int32)
    nodes32 = nodes.astype(jnp.int32)
    W1a = W1[:EMBED_DIM]
    W1b = W1[EMBED_DIM:]
    w3row = W3.reshape(1, EMBED_DIM)
    b1r = b1.reshape(1, EMBED_DIM)
    b2r = b2.reshape(1, EMBED_DIM)
    outs = []
    for s in range(NSLICE):
        nidx_s = lax.slice(neigh_idx, (s * SL_EDGES,), ((s + 1) * SL_EDGES,))
        uidx_s = jnp.pad(
            lax.slice(nodes32, (s * SL_NODES,), ((s + 1) * SL_NODES,)),
            (0, UR_PAD - SL_NODES))
        eu_s, ur_s = _sc_gather(u2e, nidx_s, uidx_s)
        outs.append(_tc_mlp(eu_s, ur_s, W1a, W1b, W2, w3row, b1r, b2r))
    return jnp.concatenate(outs, axis=0)
